# baseline (speedup 1.0000x reference)
import functools

import jax
import jax.numpy as jnp
from jax import lax
from jax.experimental import pallas as pl
from jax.experimental.pallas import tpu as pltpu
from jax.experimental.pallas import tpu_sc as plsc

N = 8192
H = 25
W = 25
C = 3
V = 7
STEPS = 5

NC = 2
NS = 16
NWORK = NC * NS
LANES = 16

ENV_BLK = 16
ENV_PER_WORKER = N // NWORK
NBLK = ENV_PER_WORKER // ENV_BLK
GWORDS = H * W * C
GROWS = 15

TCB = 512
NCHUNK = 4


def _sc_crop_body(epw, nblk, gpad, pos0, pos1, dirs, ch0, ch1, ch2,
                  grid_a, grid_b,
                  p0_v, p1_v, d_v, o0a, o1a, o2a, o0b, o1b, o2b,
                  sem_ga, sem_gb, sem_oa, sem_ob):
    wid = lax.axis_index("s") * NC + lax.axis_index("c")
    lane = lax.iota(jnp.int32, LANES)
    lane_g = lane * (GROWS * 128)
    wbase = wid * epw

    pltpu.sync_copy(pos0.at[pl.ds(wbase, epw)], p0_v)
    pltpu.sync_copy(pos1.at[pl.ds(wbase, epw)], p1_v)
    pltpu.sync_copy(dirs.at[pl.ds(wbase, epw)], d_v)

    def _grid_slice(blk):
        return gpad.at[pl.ds((wbase + blk * ENV_BLK) * GROWS, ENV_BLK * GROWS), :]

    pltpu.async_copy(_grid_slice(0), grid_a, sem_ga)
    pltpu.async_copy(_grid_slice(1), grid_b, sem_gb)

    bufs = ((grid_a, sem_ga, (o0a, o1a, o2a), sem_oa),
            (grid_b, sem_gb, (o0b, o1b, o2b), sem_ob))

    @pl.loop(0, nblk, step=2)
    def _(blk0):
        for b in range(2):
            grid_v, sem_g, (o0_v, o1_v, o2_v), sem_o = bufs[b]
            blk = blk0 + b
            base = wbase + blk * ENV_BLK
            pltpu.make_async_copy(_grid_slice(blk), grid_v, sem_g).wait()

            @pl.when(blk >= 2)
            def _():
                pltpu.make_async_copy(o0_v, ch0.at[pl.ds(base, ENV_BLK), :], sem_o).wait()
                pltpu.make_async_copy(o1_v, ch1.at[pl.ds(base, ENV_BLK), :], sem_o).wait()
                pltpu.make_async_copy(o2_v, ch2.at[pl.ds(base, ENV_BLK), :], sem_o).wait()

            p0 = p0_v[pl.ds(blk * ENV_BLK, ENV_BLK)]
            p1 = p1_v[pl.ds(blk * ENV_BLK, ENV_BLK)]
            d = d_v[pl.ds(blk * ENV_BLK, ENV_BLK)]

            off0 = jnp.where(d == 0, 0, jnp.where(d == 1, -3, jnp.where(d == 2, -6, -3)))
            off1 = jnp.where(d == 0, -3, jnp.where(d == 1, 0, jnp.where(d == 2, -3, -6)))
            top0 = p0 + off0
            top1 = p1 + off1
            fu = d <= 1
            fv = (d == 1) | (d == 2)
            tr = (d == 0) | (d == 2)

            u75 = []
            ubad = []
            vb = []
            vbad = []
            for x in range(V):
                u = top0 + jnp.where(fu, 6 - x, x)
                ubad.append((u < 0) | (u > H - 1))
                u75.append(lane_g + jnp.clip(u, 0, H - 1) * (W * C))
                v = top1 + jnp.where(fv, 6 - x, x)
                vbad.append((v < 0) | (v > W - 1))
                vb.append(jnp.clip(v, 0, W - 1))

            two = jnp.full((LANES,), 2.0, jnp.float32)
            for p in range(V * V):
                i, j = p // V, p % V
                a75 = jnp.where(tr, u75[j], u75[i])
                b = jnp.where(tr, vb[i], vb[j])
                bad = jnp.where(tr, ubad[j], ubad[i]) | jnp.where(tr, vbad[i], vbad[j])
                idx = a75 + b
                pcol = jnp.full((LANES,), p, jnp.int32)
                g0 = plsc.load_gather(grid_v, [idx >> 7, idx & 127])
                g1 = plsc.load_gather(grid_v, [(idx + W) >> 7, (idx + W) & 127])
                g2 = plsc.load_gather(grid_v, [(idx + 2 * W) >> 7, (idx + 2 * W) & 127])
                plsc.store_scatter(o0_v, [lane, pcol], jnp.where(bad, two, g0))
                plsc.store_scatter(o1_v, [lane, pcol], jnp.where(bad, two, g1))
                plsc.store_scatter(o2_v, [lane, pcol], jnp.where(bad, two, g2))

            pltpu.async_copy(o0_v, ch0.at[pl.ds(base, ENV_BLK), :], sem_o)
            pltpu.async_copy(o1_v, ch1.at[pl.ds(base, ENV_BLK), :], sem_o)
            pltpu.async_copy(o2_v, ch2.at[pl.ds(base, ENV_BLK), :], sem_o)

            @pl.when(blk + 2 < nblk)
            def _():
                pltpu.async_copy(_grid_slice(blk + 2), grid_v, sem_g)

    for b in range(2):
        _, _, (o0_v, o1_v, o2_v), sem_o = bufs[b]
        blk = nblk - 2 + b
        base = wbase + blk * ENV_BLK
        pltpu.make_async_copy(o0_v, ch0.at[pl.ds(base, ENV_BLK), :], sem_o).wait()
        pltpu.make_async_copy(o1_v, ch1.at[pl.ds(base, ENV_BLK), :], sem_o).wait()
        pltpu.make_async_copy(o2_v, ch2.at[pl.ds(base, ENV_BLK), :], sem_o).wait()


def _sc_crop(gpad, pos0, pos1, dirs, n):
    epw = n // NWORK
    nblk = epw // ENV_BLK
    mesh = plsc.VectorSubcoreMesh(core_axis_name="c", subcore_axis_name="s",
                                  num_cores=NC, num_subcores=NS)
    f = pl.kernel(
        functools.partial(_sc_crop_body, epw, nblk),
        out_type=(
            jax.ShapeDtypeStruct((n, 128), jnp.float32),
            jax.ShapeDtypeStruct((n, 128), jnp.float32),
            jax.ShapeDtypeStruct((n, 128), jnp.float32),
        ),
        mesh=mesh,
        compiler_params=pltpu.CompilerParams(needs_layout_passes=False,
                                             use_tc_tiling_on_sc=True),
        scratch_types=[
            pltpu.VMEM((ENV_BLK * GROWS, 128), jnp.float32),
            pltpu.VMEM((ENV_BLK * GROWS, 128), jnp.float32),
            pltpu.VMEM((epw,), jnp.int32),
            pltpu.VMEM((epw,), jnp.int32),
            pltpu.VMEM((epw,), jnp.int32),
            pltpu.VMEM((ENV_BLK, 128), jnp.float32),
            pltpu.VMEM((ENV_BLK, 128), jnp.float32),
            pltpu.VMEM((ENV_BLK, 128), jnp.float32),
            pltpu.VMEM((ENV_BLK, 128), jnp.float32),
            pltpu.VMEM((ENV_BLK, 128), jnp.float32),
            pltpu.VMEM((ENV_BLK, 128), jnp.float32),
            pltpu.SemaphoreType.DMA,
            pltpu.SemaphoreType.DMA,
            pltpu.SemaphoreType.DMA,
            pltpu.SemaphoreType.DMA,
        ],
    )
    return f(gpad, pos0, pos1, dirs)


TRB = 512


def _tr_body(g_ref, out_ref):
    x = g_ref[...].reshape(GWORDS, TRB)
    x = jnp.concatenate([x, jnp.zeros((GROWS * 128 - GWORDS, TRB), jnp.float32)],
                        axis=0)
    x = x.T
    out_ref[...] = x.reshape(TRB * GROWS, 128)


def _tc_transpose(gfold, chunk, n):
    grid = (n // TRB,)
    base = chunk * (n // TRB)
    return pl.pallas_call(
        _tr_body,
        grid=grid,
        in_specs=[pl.BlockSpec((H, C, W, TRB), lambda i: (0, 0, 0, base + i))],
        out_specs=pl.BlockSpec((TRB * GROWS, 128), lambda i: (i, 0)),
        out_shape=jax.ShapeDtypeStruct((n * GROWS, 128), jnp.float32),
    )(gfold)


def _tc_body(kern_ref, ch0_ref, ch1_ref, ch2_ref, out_ref):
    f32 = jnp.float32
    i32 = jnp.int32
    q = lax.broadcasted_iota(i32, (128, 128), 0)
    p = lax.broadcasted_iota(i32, (128, 128), 1)
    qi, qj = q // V, q % V
    pi, pj = p // V, p % V
    valid = (q < V * V) & (p < V * V)
    Wm = jnp.zeros((128, 128), f32)
    for dr in range(3):
        for dc in range(3):
            m = valid & (qi - pi == dr - 1) & (qj - pj == dc - 1)
            Wm = Wm + jnp.where(m, kern_ref[3 * dr + dc], 0.0)

    lanep = lax.broadcasted_iota(i32, (TCB, 128), 1)
    ok = lanep < V * V
    c0 = jnp.where(ok, ch0_ref[...], 0.0)
    c1 = jnp.where(ok, ch1_ref[...], 0.0)
    c2 = jnp.where(ok, ch2_ref[...], 0.0)

    closed = jnp.where(ok & ((c0 == 2.0) | (c2 == 1.0)), 1.0, 0.0)
    open_ = 1.0 - closed
    x = jnp.where(lanep == 27, 1.0, 0.0).astype(f32)
    for _ in range(STEPS):
        y = jnp.dot(x, Wm, preferred_element_type=f32)
        x = -0.01 * closed + jnp.tanh(y) * open_
    x = (x > 0).astype(f32)
    y = jnp.dot(x, Wm, preferred_element_type=f32)
    mask = (y > 0).astype(f32)

    def plane(mc):
        t = jnp.transpose(mc)
        return t[:V * V].reshape(V, V, TCB)
    out_ref[...] = jnp.stack([plane(mask * c0), plane(mask * c1),
                              plane(mask * c2)], axis=1)


def _tc_flood(kern9, ch0, ch1, ch2, n):
    grid = (n // TCB,)
    return pl.pallas_call(
        _tc_body,
        grid=grid,
        in_specs=[
            pl.BlockSpec(memory_space=pltpu.SMEM),
            pl.BlockSpec((TCB, 128), lambda i: (i, 0)),
            pl.BlockSpec((TCB, 128), lambda i: (i, 0)),
            pl.BlockSpec((TCB, 128), lambda i: (i, 0)),
        ],
        out_specs=pl.BlockSpec((V, C, V, TCB), lambda i: (0, 0, 0, i)),
        out_shape=jax.ShapeDtypeStruct((V, C, V, n), jnp.float32),
    )(kern9, ch0, ch1, ch2)


def kernel(grids, agent_pos, agent_dir, kernel):
    gfold = jnp.transpose(grids, (1, 3, 2, 0))
    pos0 = agent_pos[:, 0].astype(jnp.int32)
    pos1 = agent_pos[:, 1].astype(jnp.int32)
    dirs = agent_dir.astype(jnp.int32)
    kern9 = kernel.reshape(9)
    nch = N // NCHUNK
    outs = []
    for c in range(NCHUNK):
        gpad = _tc_transpose(gfold, c, nch)
        sl = pl.ds(c * nch, nch) if False else slice(c * nch, (c + 1) * nch)
        ch0, ch1, ch2 = _sc_crop(gpad, pos0[sl], pos1[sl], dirs[sl], nch)
        outs.append(_tc_flood(kern9, ch0, ch1, ch2, nch))
    out = jnp.concatenate(outs, axis=3)
    return jnp.transpose(out, (3, 0, 2, 1))


# speedup vs baseline: 1.0129x; 1.0129x over previous
import functools

import jax
import jax.numpy as jnp
from jax import lax
from jax.experimental import pallas as pl
from jax.experimental.pallas import tpu as pltpu
from jax.experimental.pallas import tpu_sc as plsc

N = 8192
H = 25
W = 25
C = 3
V = 7
STEPS = 5

NC = 2
NS = 16
NWORK = NC * NS
LANES = 16

ENV_BLK = 16
ENV_PER_WORKER = N // NWORK
NBLK = ENV_PER_WORKER // ENV_BLK
GWORDS = H * W * C
GROWS = 15

TCB = 512
NCHUNK = 2


def _sc_crop_body(epw, nblk, gpad, pos0, pos1, dirs, ch0, ch1, ch2,
                  grid_a, grid_b,
                  p0_v, p1_v, d_v, o0a, o1a, o2a, o0b, o1b, o2b,
                  sem_ga, sem_gb, sem_oa, sem_ob):
    wid = lax.axis_index("s") * NC + lax.axis_index("c")
    lane = lax.iota(jnp.int32, LANES)
    lane_g = lane * (GROWS * 128)
    wbase = wid * epw

    pltpu.sync_copy(pos0.at[pl.ds(wbase, epw)], p0_v)
    pltpu.sync_copy(pos1.at[pl.ds(wbase, epw)], p1_v)
    pltpu.sync_copy(dirs.at[pl.ds(wbase, epw)], d_v)

    def _grid_slice(blk):
        return gpad.at[pl.ds((wbase + blk * ENV_BLK) * GROWS, ENV_BLK * GROWS), :]

    pltpu.async_copy(_grid_slice(0), grid_a, sem_ga)
    pltpu.async_copy(_grid_slice(1), grid_b, sem_gb)

    bufs = ((grid_a, sem_ga, (o0a, o1a, o2a), sem_oa),
            (grid_b, sem_gb, (o0b, o1b, o2b), sem_ob))

    @pl.loop(0, nblk, step=2)
    def _(blk0):
        for b in range(2):
            grid_v, sem_g, (o0_v, o1_v, o2_v), sem_o = bufs[b]
            blk = blk0 + b
            base = wbase + blk * ENV_BLK
            pltpu.make_async_copy(_grid_slice(blk), grid_v, sem_g).wait()

            @pl.when(blk >= 2)
            def _():
                pltpu.make_async_copy(o0_v, ch0.at[pl.ds(base, ENV_BLK), :], sem_o).wait()
                pltpu.make_async_copy(o1_v, ch1.at[pl.ds(base, ENV_BLK), :], sem_o).wait()
                pltpu.make_async_copy(o2_v, ch2.at[pl.ds(base, ENV_BLK), :], sem_o).wait()

            p0 = p0_v[pl.ds(blk * ENV_BLK, ENV_BLK)]
            p1 = p1_v[pl.ds(blk * ENV_BLK, ENV_BLK)]
            d = d_v[pl.ds(blk * ENV_BLK, ENV_BLK)]

            off0 = jnp.where(d == 0, 0, jnp.where(d == 1, -3, jnp.where(d == 2, -6, -3)))
            off1 = jnp.where(d == 0, -3, jnp.where(d == 1, 0, jnp.where(d == 2, -3, -6)))
            top0 = p0 + off0
            top1 = p1 + off1
            fu = d <= 1
            fv = (d == 1) | (d == 2)
            tr = (d == 0) | (d == 2)

            u75 = []
            ubad = []
            vb = []
            vbad = []
            for x in range(V):
                u = top0 + jnp.where(fu, 6 - x, x)
                ubad.append((u < 0) | (u > H - 1))
                u75.append(lane_g + jnp.clip(u, 0, H - 1) * (W * C))
                v = top1 + jnp.where(fv, 6 - x, x)
                vbad.append((v < 0) | (v > W - 1))
                vb.append(jnp.clip(v, 0, W - 1))

            two = jnp.full((LANES,), 2.0, jnp.float32)
            for p in range(V * V):
                i, j = p // V, p % V
                a75 = jnp.where(tr, u75[j], u75[i])
                b = jnp.where(tr, vb[i], vb[j])
                bad = jnp.where(tr, ubad[j], ubad[i]) | jnp.where(tr, vbad[i], vbad[j])
                idx = a75 + b
                pcol = jnp.full((LANES,), p, jnp.int32)
                g0 = plsc.load_gather(grid_v, [idx >> 7, idx & 127])
                g1 = plsc.load_gather(grid_v, [(idx + W) >> 7, (idx + W) & 127])
                g2 = plsc.load_gather(grid_v, [(idx + 2 * W) >> 7, (idx + 2 * W) & 127])
                plsc.store_scatter(o0_v, [lane, pcol], jnp.where(bad, two, g0))
                plsc.store_scatter(o1_v, [lane, pcol], jnp.where(bad, two, g1))
                plsc.store_scatter(o2_v, [lane, pcol], jnp.where(bad, two, g2))

            pltpu.async_copy(o0_v, ch0.at[pl.ds(base, ENV_BLK), :], sem_o)
            pltpu.async_copy(o1_v, ch1.at[pl.ds(base, ENV_BLK), :], sem_o)
            pltpu.async_copy(o2_v, ch2.at[pl.ds(base, ENV_BLK), :], sem_o)

            @pl.when(blk + 2 < nblk)
            def _():
                pltpu.async_copy(_grid_slice(blk + 2), grid_v, sem_g)

    for b in range(2):
        _, _, (o0_v, o1_v, o2_v), sem_o = bufs[b]
        blk = nblk - 2 + b
        base = wbase + blk * ENV_BLK
        pltpu.make_async_copy(o0_v, ch0.at[pl.ds(base, ENV_BLK), :], sem_o).wait()
        pltpu.make_async_copy(o1_v, ch1.at[pl.ds(base, ENV_BLK), :], sem_o).wait()
        pltpu.make_async_copy(o2_v, ch2.at[pl.ds(base, ENV_BLK), :], sem_o).wait()


def _sc_crop(gpad, pos0, pos1, dirs, n):
    epw = n // NWORK
    nblk = epw // ENV_BLK
    mesh = plsc.VectorSubcoreMesh(core_axis_name="c", subcore_axis_name="s",
                                  num_cores=NC, num_subcores=NS)
    f = pl.kernel(
        functools.partial(_sc_crop_body, epw, nblk),
        out_type=(
            jax.ShapeDtypeStruct((n, 128), jnp.float32),
            jax.ShapeDtypeStruct((n, 128), jnp.float32),
            jax.ShapeDtypeStruct((n, 128), jnp.float32),
        ),
        mesh=mesh,
        compiler_params=pltpu.CompilerParams(needs_layout_passes=False,
                                             use_tc_tiling_on_sc=True),
        scratch_types=[
            pltpu.VMEM((ENV_BLK * GROWS, 128), jnp.float32),
            pltpu.VMEM((ENV_BLK * GROWS, 128), jnp.float32),
            pltpu.VMEM((epw,), jnp.int32),
            pltpu.VMEM((epw,), jnp.int32),
            pltpu.VMEM((epw,), jnp.int32),
            pltpu.VMEM((ENV_BLK, 128), jnp.float32),
            pltpu.VMEM((ENV_BLK, 128), jnp.float32),
            pltpu.VMEM((ENV_BLK, 128), jnp.float32),
            pltpu.VMEM((ENV_BLK, 128), jnp.float32),
            pltpu.VMEM((ENV_BLK, 128), jnp.float32),
            pltpu.VMEM((ENV_BLK, 128), jnp.float32),
            pltpu.SemaphoreType.DMA,
            pltpu.SemaphoreType.DMA,
            pltpu.SemaphoreType.DMA,
            pltpu.SemaphoreType.DMA,
        ],
    )
    return f(gpad, pos0, pos1, dirs)


TRB = 1024


def _tr_body(g_ref, out_ref):
    x = g_ref[...].reshape(GWORDS, TRB)
    x = jnp.concatenate([x, jnp.zeros((GROWS * 128 - GWORDS, TRB), jnp.float32)],
                        axis=0)
    x = x.T
    out_ref[...] = x.reshape(TRB * GROWS, 128)


def _tc_transpose(gfold, chunk, n):
    grid = (n // TRB,)
    base = chunk * (n // TRB)
    return pl.pallas_call(
        _tr_body,
        grid=grid,
        compiler_params=pltpu.CompilerParams(vmem_limit_bytes=56 * 1024 * 1024),
        in_specs=[pl.BlockSpec((H, C, W, TRB), lambda i: (0, 0, 0, base + i))],
        out_specs=pl.BlockSpec((TRB * GROWS, 128), lambda i: (i, 0)),
        out_shape=jax.ShapeDtypeStruct((n * GROWS, 128), jnp.float32),
    )(gfold)


def _tc_body(kern_ref, ch0_ref, ch1_ref, ch2_ref, out_ref):
    f32 = jnp.float32
    i32 = jnp.int32
    q = lax.broadcasted_iota(i32, (128, 128), 0)
    p = lax.broadcasted_iota(i32, (128, 128), 1)
    qi, qj = q // V, q % V
    pi, pj = p // V, p % V
    valid = (q < V * V) & (p < V * V)
    Wm = jnp.zeros((128, 128), f32)
    for dr in range(3):
        for dc in range(3):
            m = valid & (qi - pi == dr - 1) & (qj - pj == dc - 1)
            Wm = Wm + jnp.where(m, kern_ref[3 * dr + dc], 0.0)

    lanep = lax.broadcasted_iota(i32, (TCB, 128), 1)
    ok = lanep < V * V
    c0 = jnp.where(ok, ch0_ref[...], 0.0)
    c1 = jnp.where(ok, ch1_ref[...], 0.0)
    c2 = jnp.where(ok, ch2_ref[...], 0.0)

    closed = jnp.where(ok & ((c0 == 2.0) | (c2 == 1.0)), 1.0, 0.0)
    open_ = 1.0 - closed
    x = jnp.where(lanep == 27, 1.0, 0.0).astype(f32)
    for _ in range(STEPS):
        y = jnp.dot(x, Wm, preferred_element_type=f32)
        x = -0.01 * closed + jnp.tanh(y) * open_
    x = (x > 0).astype(f32)
    y = jnp.dot(x, Wm, preferred_element_type=f32)
    mask = (y > 0).astype(f32)

    def plane(mc):
        t = jnp.transpose(mc)
        return t[:V * V].reshape(V, V, TCB)
    out_ref[...] = jnp.stack([plane(mask * c0), plane(mask * c1),
                              plane(mask * c2)], axis=1)


def _tc_flood(kern9, ch0, ch1, ch2, n):
    grid = (n // TCB,)
    return pl.pallas_call(
        _tc_body,
        grid=grid,
        in_specs=[
            pl.BlockSpec(memory_space=pltpu.SMEM),
            pl.BlockSpec((TCB, 128), lambda i: (i, 0)),
            pl.BlockSpec((TCB, 128), lambda i: (i, 0)),
            pl.BlockSpec((TCB, 128), lambda i: (i, 0)),
        ],
        out_specs=pl.BlockSpec((V, C, V, TCB), lambda i: (0, 0, 0, i)),
        out_shape=jax.ShapeDtypeStruct((V, C, V, n), jnp.float32),
    )(kern9, ch0, ch1, ch2)


def kernel(grids, agent_pos, agent_dir, kernel):
    gfold = jnp.transpose(grids, (1, 3, 2, 0))
    pos0 = agent_pos[:, 0].astype(jnp.int32)
    pos1 = agent_pos[:, 1].astype(jnp.int32)
    dirs = agent_dir.astype(jnp.int32)
    kern9 = kernel.reshape(9)
    nch = N // NCHUNK
    outs = []
    for c in range(NCHUNK):
        gpad = _tc_transpose(gfold, c, nch)
        sl = pl.ds(c * nch, nch) if False else slice(c * nch, (c + 1) * nch)
        ch0, ch1, ch2 = _sc_crop(gpad, pos0[sl], pos1[sl], dirs[sl], nch)
        outs.append(_tc_flood(kern9, ch0, ch1, ch2, nch))
    out = jnp.concatenate(outs, axis=3)
    return jnp.transpose(out, (3, 0, 2, 1))


# speedup vs baseline: 1.0983x; 1.0843x over previous
import functools

import jax
import jax.numpy as jnp
from jax import lax
from jax.experimental import pallas as pl
from jax.experimental.pallas import tpu as pltpu
from jax.experimental.pallas import tpu_sc as plsc

N = 8192
H = 25
W = 25
C = 3
V = 7
STEPS = 5

NC = 2
NS = 16
NWORK = NC * NS
LANES = 16

ENV_BLK = 16
ENV_PER_WORKER = N // NWORK
NBLK = ENV_PER_WORKER // ENV_BLK
GWORDS = H * W * C
GROWS = 15
GB_ROWS = 6

TCB = 512
NCHUNK = 2


def _sc_crop_body(epw, nblk, tot_rows, gpad, pos0, pos1, dirs, ch0, ch1, ch2,
                  grid_a, grid_b, idx_a, idx_b,
                  p0_v, p1_v, d_v, o0a, o1a, o2a, o0b, o1b, o2b,
                  sem_ga, sem_gb, sem_oa, sem_ob):
    wid = lax.axis_index("s") * NC + lax.axis_index("c")
    lane = lax.iota(jnp.int32, LANES)
    lane6 = lane * GB_ROWS
    wbase = wid * epw

    pltpu.sync_copy(pos0.at[pl.ds(wbase, epw)], p0_v)
    pltpu.sync_copy(pos1.at[pl.ds(wbase, epw)], p1_v)
    pltpu.sync_copy(dirs.at[pl.ds(wbase, epw)], d_v)

    def _band_r0(blk):
        p0b = p0_v[pl.ds(blk * ENV_BLK, ENV_BLK)]
        db = d_v[pl.ds(blk * ENV_BLK, ENV_BLK)]
        off0b = jnp.where(db == 0, 0,
                          jnp.where(db == 1, -3, jnp.where(db == 2, -6, -3)))
        astart = jnp.clip(p0b + off0b, 0, H - V)
        e = wbase + blk * ENV_BLK + lane
        return (e * (GROWS * 128) + astart * (W * C)) >> 7

    def _fire(blk, grid_v, idx_v, sem):
        r0 = _band_r0(blk)
        for s in range(GB_ROWS):
            r = r0 + s
            if s == GB_ROWS - 1:
                r = jnp.minimum(r, tot_rows - 1)
            plsc.store_scatter(idx_v, [lane6 + s], r)
        pltpu.async_copy(gpad.at[idx_v], grid_v, sem)

    _fire(0, grid_a, idx_a, sem_ga)
    _fire(1, grid_b, idx_b, sem_gb)

    bufs = ((grid_a, sem_ga, idx_a, (o0a, o1a, o2a), sem_oa),
            (grid_b, sem_gb, idx_b, (o0b, o1b, o2b), sem_ob))

    @pl.loop(0, nblk, step=2)
    def _(blk0):
        for b in range(2):
            grid_v, sem_g, idx_v, (o0_v, o1_v, o2_v), sem_o = bufs[b]
            blk = blk0 + b
            base = wbase + blk * ENV_BLK
            pltpu.make_async_copy(gpad.at[idx_v], grid_v, sem_g).wait()

            @pl.when(blk >= 2)
            def _():
                pltpu.make_async_copy(o0_v, ch0.at[pl.ds(base, ENV_BLK), :], sem_o).wait()
                pltpu.make_async_copy(o1_v, ch1.at[pl.ds(base, ENV_BLK), :], sem_o).wait()
                pltpu.make_async_copy(o2_v, ch2.at[pl.ds(base, ENV_BLK), :], sem_o).wait()

            p0 = p0_v[pl.ds(blk * ENV_BLK, ENV_BLK)]
            p1 = p1_v[pl.ds(blk * ENV_BLK, ENV_BLK)]
            d = d_v[pl.ds(blk * ENV_BLK, ENV_BLK)]

            off0 = jnp.where(d == 0, 0, jnp.where(d == 1, -3, jnp.where(d == 2, -6, -3)))
            off1 = jnp.where(d == 0, -3, jnp.where(d == 1, 0, jnp.where(d == 2, -3, -6)))
            top0 = p0 + off0
            top1 = p1 + off1
            fu = d <= 1
            fv = (d == 1) | (d == 2)
            tr = (d == 0) | (d == 2)

            e_words = (wbase + blk * ENV_BLK + lane) * (GROWS * 128)
            r0v = _band_r0(blk) - lane6

            u75 = []
            ubad = []
            vb = []
            vbad = []
            for x in range(V):
                u = top0 + jnp.where(fu, 6 - x, x)
                ubad.append((u < 0) | (u > H - 1))
                u75.append(e_words + jnp.clip(u, 0, H - 1) * (W * C))
                v = top1 + jnp.where(fv, 6 - x, x)
                vbad.append((v < 0) | (v > W - 1))
                vb.append(jnp.clip(v, 0, W - 1))

            two = jnp.full((LANES,), 2.0, jnp.float32)
            for p in range(V * V):
                i, j = p // V, p % V
                a75 = jnp.where(tr, u75[j], u75[i])
                b = jnp.where(tr, vb[i], vb[j])
                bad = jnp.where(tr, ubad[j], ubad[i]) | jnp.where(tr, vbad[i], vbad[j])
                idx = a75 + b
                pcol = jnp.full((LANES,), p, jnp.int32)
                g0 = plsc.load_gather(grid_v, [(idx >> 7) - r0v, idx & 127])
                g1 = plsc.load_gather(grid_v, [((idx + W) >> 7) - r0v, (idx + W) & 127])
                g2 = plsc.load_gather(
                    grid_v, [((idx + 2 * W) >> 7) - r0v, (idx + 2 * W) & 127])
                plsc.store_scatter(o0_v, [lane, pcol], jnp.where(bad, two, g0))
                plsc.store_scatter(o1_v, [lane, pcol], jnp.where(bad, two, g1))
                plsc.store_scatter(o2_v, [lane, pcol], jnp.where(bad, two, g2))

            pltpu.async_copy(o0_v, ch0.at[pl.ds(base, ENV_BLK), :], sem_o)
            pltpu.async_copy(o1_v, ch1.at[pl.ds(base, ENV_BLK), :], sem_o)
            pltpu.async_copy(o2_v, ch2.at[pl.ds(base, ENV_BLK), :], sem_o)

            @pl.when(blk + 2 < nblk)
            def _():
                _fire(blk + 2, grid_v, idx_v, sem_g)

    for b in range(2):
        _, _, _, (o0_v, o1_v, o2_v), sem_o = bufs[b]
        blk = nblk - 2 + b
        base = wbase + blk * ENV_BLK
        pltpu.make_async_copy(o0_v, ch0.at[pl.ds(base, ENV_BLK), :], sem_o).wait()
        pltpu.make_async_copy(o1_v, ch1.at[pl.ds(base, ENV_BLK), :], sem_o).wait()
        pltpu.make_async_copy(o2_v, ch2.at[pl.ds(base, ENV_BLK), :], sem_o).wait()


def _sc_crop(gpad, pos0, pos1, dirs, n):
    epw = n // NWORK
    nblk = epw // ENV_BLK
    mesh = plsc.VectorSubcoreMesh(core_axis_name="c", subcore_axis_name="s",
                                  num_cores=NC, num_subcores=NS)
    f = pl.kernel(
        functools.partial(_sc_crop_body, epw, nblk, n * GROWS),
        out_type=(
            jax.ShapeDtypeStruct((n, 128), jnp.float32),
            jax.ShapeDtypeStruct((n, 128), jnp.float32),
            jax.ShapeDtypeStruct((n, 128), jnp.float32),
        ),
        mesh=mesh,
        compiler_params=pltpu.CompilerParams(needs_layout_passes=False,
                                             use_tc_tiling_on_sc=True),
        scratch_types=[
            pltpu.VMEM((ENV_BLK * GB_ROWS, 128), jnp.float32),
            pltpu.VMEM((ENV_BLK * GB_ROWS, 128), jnp.float32),
            pltpu.VMEM((ENV_BLK * GB_ROWS,), jnp.int32),
            pltpu.VMEM((ENV_BLK * GB_ROWS,), jnp.int32),
            pltpu.VMEM((epw,), jnp.int32),
            pltpu.VMEM((epw,), jnp.int32),
            pltpu.VMEM((epw,), jnp.int32),
            pltpu.VMEM((ENV_BLK, 128), jnp.float32),
            pltpu.VMEM((ENV_BLK, 128), jnp.float32),
            pltpu.VMEM((ENV_BLK, 128), jnp.float32),
            pltpu.VMEM((ENV_BLK, 128), jnp.float32),
            pltpu.VMEM((ENV_BLK, 128), jnp.float32),
            pltpu.VMEM((ENV_BLK, 128), jnp.float32),
            pltpu.SemaphoreType.DMA,
            pltpu.SemaphoreType.DMA,
            pltpu.SemaphoreType.DMA,
            pltpu.SemaphoreType.DMA,
        ],
    )
    return f(gpad, pos0, pos1, dirs)


TRB = 512


def _tr_body(g_ref, out_ref):
    x = g_ref[...].reshape(GWORDS, TRB)
    x = jnp.concatenate([x, jnp.zeros((GROWS * 128 - GWORDS, TRB), jnp.float32)],
                        axis=0)
    x = x.T
    out_ref[...] = x.reshape(TRB * GROWS, 128)


def _tc_transpose(gfold, chunk, n):
    grid = (n // TRB,)
    base = chunk * (n // TRB)
    return pl.pallas_call(
        _tr_body,
        grid=grid,
        in_specs=[pl.BlockSpec((H, C, W, TRB), lambda i: (0, 0, 0, base + i))],
        out_specs=pl.BlockSpec((TRB * GROWS, 128), lambda i: (i, 0)),
        out_shape=jax.ShapeDtypeStruct((n * GROWS, 128), jnp.float32),
    )(gfold)


def _tc_body(kern_ref, ch0_ref, ch1_ref, ch2_ref, out_ref):
    f32 = jnp.float32
    i32 = jnp.int32
    q = lax.broadcasted_iota(i32, (128, 128), 0)
    p = lax.broadcasted_iota(i32, (128, 128), 1)
    qi, qj = q // V, q % V
    pi, pj = p // V, p % V
    valid = (q < V * V) & (p < V * V)
    Wm = jnp.zeros((128, 128), f32)
    for dr in range(3):
        for dc in range(3):
            m = valid & (qi - pi == dr - 1) & (qj - pj == dc - 1)
            Wm = Wm + jnp.where(m, kern_ref[3 * dr + dc], 0.0)

    lanep = lax.broadcasted_iota(i32, (TCB, 128), 1)
    ok = lanep < V * V
    c0 = jnp.where(ok, ch0_ref[...], 0.0)
    c1 = jnp.where(ok, ch1_ref[...], 0.0)
    c2 = jnp.where(ok, ch2_ref[...], 0.0)

    closed = jnp.where(ok & ((c0 == 2.0) | (c2 == 1.0)), 1.0, 0.0)
    open_ = 1.0 - closed
    x = jnp.where(lanep == 27, 1.0, 0.0).astype(f32)
    for _ in range(STEPS):
        y = jnp.dot(x, Wm, preferred_element_type=f32)
        x = -0.01 * closed + jnp.tanh(y) * open_
    x = (x > 0).astype(f32)
    y = jnp.dot(x, Wm, preferred_element_type=f32)
    mask = (y > 0).astype(f32)

    def plane(mc):
        t = jnp.transpose(mc)
        return t[:V * V].reshape(V, V, TCB)
    out_ref[...] = jnp.stack([plane(mask * c0), plane(mask * c1),
                              plane(mask * c2)], axis=1)


def _tc_flood(kern9, ch0, ch1, ch2, n):
    grid = (n // TCB,)
    return pl.pallas_call(
        _tc_body,
        grid=grid,
        in_specs=[
            pl.BlockSpec(memory_space=pltpu.SMEM),
            pl.BlockSpec((TCB, 128), lambda i: (i, 0)),
            pl.BlockSpec((TCB, 128), lambda i: (i, 0)),
            pl.BlockSpec((TCB, 128), lambda i: (i, 0)),
        ],
        out_specs=pl.BlockSpec((V, C, V, TCB), lambda i: (0, 0, 0, i)),
        out_shape=jax.ShapeDtypeStruct((V, C, V, n), jnp.float32),
    )(kern9, ch0, ch1, ch2)


def kernel(grids, agent_pos, agent_dir, kernel):
    gfold = jnp.transpose(grids, (1, 3, 2, 0))
    pos0 = agent_pos[:, 0].astype(jnp.int32)
    pos1 = agent_pos[:, 1].astype(jnp.int32)
    dirs = agent_dir.astype(jnp.int32)
    kern9 = kernel.reshape(9)
    nch = N // NCHUNK
    outs = []
    for c in range(NCHUNK):
        gpad = _tc_transpose(gfold, c, nch)
        sl = pl.ds(c * nch, nch) if False else slice(c * nch, (c + 1) * nch)
        ch0, ch1, ch2 = _sc_crop(gpad, pos0[sl], pos1[sl], dirs[sl], nch)
        outs.append(_tc_flood(kern9, ch0, ch1, ch2, nch))
    out = jnp.concatenate(outs, axis=3)
    return jnp.transpose(out, (3, 0, 2, 1))
